# Initial kernel scaffold; baseline (speedup 1.0000x reference)
import jax
import jax.numpy as jnp
from jax.experimental import pallas as pl


def kernel(pos, table):
    raise NotImplementedError("write your pallas kernel here")


# speedup vs baseline: 10.9132x; 10.9132x over previous
import functools

import jax
import jax.numpy as jnp
from jax import lax
from jax.experimental import pallas as pl
from jax.experimental.pallas import tpu as pltpu
from jax.experimental.pallas import tpu_sc as plsc

D_MODEL_K = 64
MAX_LEN_K = 13941
ROWS_PAD = 13952

NC = 2
NS = 16
NW = NC * NS
CHUNK = 512


def _blur_body(a0, a1, a2, out):
    out[...] = 0.5 * a0[...] + a1[...] + 0.5 * a2[...]


def _blur(a0, a1, a2):
    shp = jax.ShapeDtypeStruct(a0.shape, jnp.float32)
    return pl.pallas_call(_blur_body, out_shape=shp)(a0, a1, a2)


def _make_gather(n_idx):
    b_per_w = n_idx // NW
    n_chunks = b_per_w // CHUNK
    mesh = plsc.VectorSubcoreMesh(core_axis_name="c", subcore_axis_name="s")

    @functools.partial(
        pl.kernel,
        mesh=mesh,
        compiler_params=pltpu.CompilerParams(use_tc_tiling_on_sc=False),
        out_type=jax.ShapeDtypeStruct((n_idx, D_MODEL_K), jnp.float32),
        scratch_types=[
            pltpu.VMEM((CHUNK,), jnp.int32),
            pltpu.VMEM((CHUNK, D_MODEL_K), jnp.float32),
            pltpu.SemaphoreType.DMA,
        ],
    )
    def gather_k(table_hbm, idx_hbm, out_hbm, idx_v, rows_v, sem):
        wid = lax.axis_index("s") * NC + lax.axis_index("c")
        base = wid * b_per_w

        def chunk_body(i, carry):
            off = base + i * CHUNK
            pltpu.sync_copy(idx_hbm.at[pl.ds(off, CHUNK)], idx_v)
            pltpu.async_copy(table_hbm.at[idx_v], rows_v, sem).wait()
            pltpu.sync_copy(rows_v, out_hbm.at[pl.ds(off, CHUNK)])
            return carry

        lax.fori_loop(0, n_chunks, chunk_body, 0)

    return gather_k


def kernel(pos, table):
    t = table.astype(jnp.float32)
    a0 = jnp.concatenate([t[0:1], t[: MAX_LEN_K - 1]], axis=0)
    a1 = t[:MAX_LEN_K]
    a2 = t[1 : MAX_LEN_K + 1]
    pad = ROWS_PAD - MAX_LEN_K
    a0, a1, a2 = (
        jnp.pad(x, ((0, pad), (0, 0))).reshape(ROWS_PAD * D_MODEL_K // 128, 128)
        for x in (a0, a1, a2)
    )
    blurred = _blur(a0, a1, a2).reshape(ROWS_PAD, D_MODEL_K)

    b, s = pos.shape
    idx = pos.reshape(-1).astype(jnp.int32)
    out = _make_gather(b * s)(blurred, idx)
    return out.reshape(b, s, D_MODEL_K)
